# Initial kernel scaffold; baseline (speedup 1.0000x reference)
import jax
import jax.numpy as jnp
from jax.experimental import pallas as pl


def kernel(x, timestamps, pe, hour_emb, day_emb, month_emb, season_emb):
    raise NotImplementedError("write your pallas kernel here")


# speedup vs baseline: 4.6039x; 4.6039x over previous
import jax
import jax.numpy as jnp
from jax import lax
from jax.experimental import pallas as pl


def _body(ts_ref, x_ref, pe_ref, emb_ref, out_ref):
    S = x_ref.shape[1]
    D = x_ref.shape[2]
    C = D // 4
    ts = ts_ref[0]
    xb = x_ref[0]
    peb = pe_ref[...]
    for c in range(4):
        idx = ts[c, :]
        oh = (idx[:, None] == lax.broadcasted_iota(jnp.int32, (S, 32), 1))
        chunk = jnp.dot(oh.astype(jnp.float32),
                        emb_ref[:, c * C:(c + 1) * C],
                        preferred_element_type=jnp.float32)
        out_ref[0, :, c * C:(c + 1) * C] = (
            xb[:, c * C:(c + 1) * C] + peb[:, c * C:(c + 1) * C] + chunk)


def kernel(x, timestamps, pe, hour_emb, day_emb, month_emb, season_emb):
    B, L, D = x.shape
    C = D // 4
    S = 256
    nsb = L // S

    pe2 = pe[0]
    tsT = timestamps.transpose(0, 2, 1)

    def pad32(e):
        return jnp.pad(e, ((0, 32 - e.shape[0]), (0, 0)))

    emb = jnp.concatenate(
        [pad32(hour_emb), pad32(day_emb), pad32(month_emb), pad32(season_emb)],
        axis=1)

    return pl.pallas_call(
        _body,
        grid=(nsb, B),
        in_specs=[
            pl.BlockSpec((1, 4, S), lambda i, j: (j, 0, i)),
            pl.BlockSpec((1, S, D), lambda i, j: (j, i, 0)),
            pl.BlockSpec((S, D), lambda i, j: (i, 0)),
            pl.BlockSpec((32, D), lambda i, j: (0, 0)),
        ],
        out_specs=pl.BlockSpec((1, S, D), lambda i, j: (j, i, 0)),
        out_shape=jax.ShapeDtypeStruct((B, L, D), x.dtype),
    )(tsT, x, pe2, emb)


# speedup vs baseline: 6.1191x; 1.3291x over previous
import jax
import jax.numpy as jnp
from jax import lax
from jax.experimental import pallas as pl


def _body(ts_ref, x_ref, pe_ref, emb_ref, out_ref):
    S = x_ref.shape[1]
    D = x_ref.shape[2]
    C = D // 4
    ts = ts_ref[0]
    xb = x_ref[0]
    peb = pe_ref[...]
    for c in range(4):
        idx = ts[c, :]
        oh = (idx[:, None] == lax.broadcasted_iota(jnp.int32, (S, 32), 1))
        chunk = jnp.dot(oh.astype(jnp.float32),
                        emb_ref[:, c * C:(c + 1) * C],
                        preferred_element_type=jnp.float32)
        out_ref[0, :, c * C:(c + 1) * C] = (
            xb[:, c * C:(c + 1) * C] + peb[:, c * C:(c + 1) * C] + chunk)


def kernel(x, timestamps, pe, hour_emb, day_emb, month_emb, season_emb):
    B, L, D = x.shape
    C = D // 4
    S = 512
    nsb = L // S

    pe2 = pe[0]
    tsT = timestamps.transpose(0, 2, 1)

    def pad32(e):
        return jnp.pad(e, ((0, 32 - e.shape[0]), (0, 0)))

    emb = jnp.concatenate(
        [pad32(hour_emb), pad32(day_emb), pad32(month_emb), pad32(season_emb)],
        axis=1)

    return pl.pallas_call(
        _body,
        grid=(nsb, B),
        in_specs=[
            pl.BlockSpec((1, 4, S), lambda i, j: (j, 0, i)),
            pl.BlockSpec((1, S, D), lambda i, j: (j, i, 0)),
            pl.BlockSpec((S, D), lambda i, j: (i, 0)),
            pl.BlockSpec((32, D), lambda i, j: (0, 0)),
        ],
        out_specs=pl.BlockSpec((1, S, D), lambda i, j: (j, i, 0)),
        out_shape=jax.ShapeDtypeStruct((B, L, D), x.dtype),
    )(tsT, x, pe2, emb)


# speedup vs baseline: 6.6761x; 1.0910x over previous
import jax
import jax.numpy as jnp
from jax import lax
from jax.experimental import pallas as pl


def _body(ts_ref, x_ref, pe_ref, emb_ref, out_ref):
    S = x_ref.shape[1]
    D = x_ref.shape[2]
    C = D // 4
    ts = ts_ref[0]
    xb = x_ref[0]
    peb = pe_ref[...]
    for c in range(4):
        idx = ts[c, :]
        oh = (idx[:, None] == lax.broadcasted_iota(jnp.int32, (S, 32), 1))
        chunk = jnp.dot(oh.astype(jnp.float32),
                        emb_ref[:, c * C:(c + 1) * C],
                        preferred_element_type=jnp.float32)
        out_ref[0, :, c * C:(c + 1) * C] = (
            xb[:, c * C:(c + 1) * C] + peb[:, c * C:(c + 1) * C] + chunk)


def kernel(x, timestamps, pe, hour_emb, day_emb, month_emb, season_emb):
    B, L, D = x.shape
    C = D // 4
    S = 1024
    nsb = L // S

    pe2 = pe[0]
    tsT = timestamps.transpose(0, 2, 1)

    def pad32(e):
        return jnp.pad(e, ((0, 32 - e.shape[0]), (0, 0)))

    emb = jnp.concatenate(
        [pad32(hour_emb), pad32(day_emb), pad32(month_emb), pad32(season_emb)],
        axis=1)

    return pl.pallas_call(
        _body,
        grid=(nsb, B),
        in_specs=[
            pl.BlockSpec((1, 4, S), lambda i, j: (j, 0, i)),
            pl.BlockSpec((1, S, D), lambda i, j: (j, i, 0)),
            pl.BlockSpec((S, D), lambda i, j: (i, 0)),
            pl.BlockSpec((32, D), lambda i, j: (0, 0)),
        ],
        out_specs=pl.BlockSpec((1, S, D), lambda i, j: (j, i, 0)),
        out_shape=jax.ShapeDtypeStruct((B, L, D), x.dtype),
    )(tsT, x, pe2, emb)


# speedup vs baseline: 7.4709x; 1.1190x over previous
import jax
import jax.numpy as jnp
from jax import lax
from jax.experimental import pallas as pl


def _body(ts_ref, x_ref, pe_ref, emb_ref, out_ref):
    S = x_ref.shape[1]
    D = x_ref.shape[2]
    C = D // 4
    ts = ts_ref[0]
    xb = x_ref[0]
    peb = pe_ref[...]
    for c in range(4):
        idx = ts[c, :]
        oh = (idx[:, None] == lax.broadcasted_iota(jnp.int32, (S, 32), 1))
        chunk = jnp.dot(oh.astype(jnp.float32),
                        emb_ref[:, c * C:(c + 1) * C],
                        preferred_element_type=jnp.float32)
        out_ref[0, :, c * C:(c + 1) * C] = (
            xb[:, c * C:(c + 1) * C] + peb[:, c * C:(c + 1) * C] + chunk)


def kernel(x, timestamps, pe, hour_emb, day_emb, month_emb, season_emb):
    B, L, D = x.shape
    C = D // 4
    S = 2048
    nsb = L // S

    pe2 = pe[0]
    tsT = timestamps.transpose(0, 2, 1)

    def pad32(e):
        return jnp.pad(e, ((0, 32 - e.shape[0]), (0, 0)))

    emb = jnp.concatenate(
        [pad32(hour_emb), pad32(day_emb), pad32(month_emb), pad32(season_emb)],
        axis=1)

    return pl.pallas_call(
        _body,
        grid=(nsb, B),
        in_specs=[
            pl.BlockSpec((1, 4, S), lambda i, j: (j, 0, i)),
            pl.BlockSpec((1, S, D), lambda i, j: (j, i, 0)),
            pl.BlockSpec((S, D), lambda i, j: (i, 0)),
            pl.BlockSpec((32, D), lambda i, j: (0, 0)),
        ],
        out_specs=pl.BlockSpec((1, S, D), lambda i, j: (j, i, 0)),
        out_shape=jax.ShapeDtypeStruct((B, L, D), x.dtype),
    )(tsT, x, pe2, emb)
